# baseline (speedup 1.0000x reference)
import jax
import jax.numpy as jnp
from jax import lax
from jax.experimental import pallas as pl


def _pos_kernel(row_ref, col_ref, out_ref):
    H, W, D = 32, 32, 256
    HW = H * W
    col = col_ref[0:W, :]
    row = row_ref[0:H, :]
    j_w = lax.broadcasted_iota(jnp.int32, (W, HW), 1)
    sel = lax.broadcasted_iota(jnp.int32, (W, HW), 0)
    T = (j_w % W == sel).astype(jnp.float32)
    R = (j_w // W == sel).astype(jnp.float32)
    dn = (((0,), (0,)), ((), ()))
    x_flat = lax.dot_general(col, T, dn, preferred_element_type=jnp.float32)
    y_flat = lax.dot_general(row, R, dn, preferred_element_type=jnp.float32)
    out_ref[...] = jnp.concatenate([x_flat, y_flat], axis=0)


def kernel(row_embeddings, column_embeddings, x):
    batch, _, height, width = x.shape
    D = row_embeddings.shape[1]
    C = 2 * D
    HW = height * width
    out = pl.pallas_call(
        _pos_kernel,
        grid=(batch,),
        in_specs=[
            pl.BlockSpec(row_embeddings.shape, lambda b: (0, 0)),
            pl.BlockSpec(column_embeddings.shape, lambda b: (0, 0)),
        ],
        out_specs=pl.BlockSpec((C, HW), lambda b: (b, 0)),
        out_shape=jax.ShapeDtypeStruct((batch * C, HW), jnp.float32),
    )(row_embeddings, column_embeddings)
    return out.reshape(batch, C, height, width)
